# baseline (speedup 1.0000x reference)
import functools

import numpy as _np
import jax
import jax.numpy as jnp
from jax import lax
from jax.experimental import pallas as pl
from jax.experimental.pallas import tpu as pltpu
from jax.experimental.pallas import tpu_sc as plsc

_LN2 = 0.6931471805599453
_LOG2E = 1.4426950408889634
_KNOTS = 512


def _exp_poly(t):
    t = jnp.maximum(t, -87.0)
    z = t * _LOG2E
    ni = (z - 0.5).astype(jnp.int32)
    g = (z - ni.astype(jnp.float32)) * _LN2
    p = 1.0 + g * (1.0 + g * (0.5 + g * (1.0 / 6.0 + g * (
        1.0 / 24.0 + g * (1.0 / 120.0 + g * (1.0 / 720.0))))))
    scale = lax.bitcast_convert_type((ni + 127) << 23, jnp.float32)
    return p * scale


def _log1p_poly(w):
    den = 2.0 + w
    r = jnp.full_like(w, 0.4)
    for _ in range(4):
        r = r * (2.0 - den * r)
    s = w * r
    s2 = s * s
    return 2.0 * s * (1.0 + s2 * (1.0 / 3.0 + s2 * (
        1.0 / 5.0 + s2 * (1.0 / 7.0))))


def _ssp(x):
    return jnp.maximum(x, 0.0) + _log1p_poly(_exp_poly(-jnp.abs(x))) - _LN2


def _xy_body(zb_ref, Ehi_ref, Elo_ref, Wi2f_ref, x_ref, y_ref):
    z = zb_ref[0, 0, :]
    A = z.shape[0]
    Z = (z[:, None] == lax.broadcasted_iota(jnp.int32, (A, 128), 1))
    Zb = Z.astype(jnp.bfloat16)
    x = (jnp.dot(Zb, Ehi_ref[...], preferred_element_type=jnp.float32)
         + jnp.dot(Zb, Elo_ref[...], preferred_element_type=jnp.float32))
    x_ref[0] = x
    y_ref[0] = jnp.dot(x.astype(jnp.bfloat16), Wi2f_ref[...],
                       preferred_element_type=jnp.float32).astype(jnp.bfloat16)


def _tab_body(W1_ref, b1_ref, W2_ref, b2_ref, offs_ref, tab_ref, *, coeff, hu, Kp):
    k = lax.broadcasted_iota(jnp.int32, (Kp, 1), 0).astype(jnp.float32)
    r = jnp.sqrt(k * hu + 1e-12)
    fg = _exp_poly(coeff * (r - offs_ref[...]) ** 2)
    h = _ssp(lax.dot_general(fg, W1_ref[...], (((1,), (0,)), ((), ())),
                             precision=lax.Precision.HIGHEST) + b1_ref[...])
    wf = lax.dot_general(h, W2_ref[...], (((1,), (0,)), ((), ())),
                         precision=lax.Precision.HIGHEST) + b2_ref[...]
    tab_ref[...] = wf.astype(jnp.bfloat16)


def _vsplat(v, k):
    idx = jnp.full((16, 1), k, dtype=jnp.int32)
    dn = lax.GatherDimensionNumbers(offset_dims=(), collapsed_slice_dims=(0,),
                                    start_index_map=(0,))
    return lax.gather(v, idx, dn, (1,),
                      mode=lax.GatherScatterMode.PROMISE_IN_BOUNDS)


def _sc_body(ypack, wtab, posx, posy, posz, nbh, mask, out, ytab_v, wtab_v,
             px_v, py_v, pz_v, nbh_v, mask_v, out_v, cb_y, cb_t, cb_g, cb_f, *, A, N, BA, K, inv_hu,
             atoms_per_tile, chunk, NWG):
    NC = 2
    cid = lax.axis_index("c")
    sid = lax.axis_index("s")
    wid = sid * NC + cid
    b = wid // 4
    sub = wid % 4
    atom0 = (sub // 2) * atoms_per_tile
    ch = sub % 2
    HW = NWG * 16
    NPAD = N + 16

    pltpu.sync_copy(wtab.at[pl.ds(ch * ((K + 8) * HW), (K + 8) * HW)], wtab_v)
    pltpu.sync_copy(ypack.at[pl.ds((ch * BA + b * A) * HW, A * HW)], ytab_v)
    pltpu.sync_copy(posx.at[pl.ds(b * A, A)], px_v)
    pltpu.sync_copy(posy.at[pl.ds(b * A, A)], py_v)
    pltpu.sync_copy(posz.at[pl.ds(b * A, A)], pz_v)

    row0 = b * A + atom0
    iota16 = lax.broadcasted_iota(jnp.int32, (16,), 0)
    cols = [iota16 + 16 * w for w in range(NWG)]
    nq = N // 16

    for c in range(atoms_per_tile // chunk):
        crow = row0 + c * chunk
        pltpu.sync_copy(nbh.at[pl.ds(crow * N, chunk * N)], nbh_v)
        pltpu.sync_copy(mask.at[pl.ds(crow * N, chunk * N)], mask_v)

        def atom_body(ai, carry):
            a_loc = atom0 + c * chunk + ai
            af = jnp.full((16,), a_loc, dtype=jnp.int32)
            pxa = plsc.load_gather(px_v, [af])
            pya = plsc.load_gather(py_v, [af])
            pza = plsc.load_gather(pz_v, [af])
            zi = jnp.zeros((16,), jnp.int32)
            zf = jnp.zeros((16,), jnp.float32)
            for g in range(NPAD // 16):
                cb_y[pl.ds(16 * g, 16)] = zi
                cb_t[pl.ds(16 * g, 16)] = zi
                cb_g[pl.ds(16 * g, 16)] = zf
                cb_f[pl.ds(16 * g, 16)] = zf
            n_act = 0
            for q in range(nq):
                nb16 = nbh_v[pl.ds(ai * N + q * 16, 16)]
                m16 = mask_v[pl.ds(ai * N + q * 16, 16)]
                dx = plsc.load_gather(px_v, [nb16]) - pxa
                dy = plsc.load_gather(py_v, [nb16]) - pya
                dz = plsc.load_gather(pz_v, [nb16]) - pza
                u = dx * dx + dy * dy + dz * dz
                act = u + 1e-12 <= 25.0
                gate = jnp.where(act, m16, 0.0)
                t = u * inv_hu
                i016 = jnp.clip(t.astype(jnp.int32), 0, K - 1)
                fr16 = t - i016.astype(jnp.float32)
                fg16 = fr16 * gate
                off = pl.multiple_of(n_act, 1)
                plsc.store_compressed(cb_y.at[pl.ds(off, 16)], nb16 * HW, mask=act)
                plsc.store_compressed(cb_t.at[pl.ds(off, 16)], i016 * HW, mask=act)
                plsc.store_compressed(cb_g.at[pl.ds(off, 16)], gate, mask=act)
                plsc.store_compressed(cb_f.at[pl.ds(off, 16)], fg16, mask=act)
                n_act = n_act + plsc.all_reduce_population_count(act)[0]
            acc = [jnp.zeros((16,), jnp.float32) for _ in range(2 * NWG)]

            def group_body(g, acc):
                base = pl.multiple_of(g * 16, 16)
                ybase16 = cb_y[pl.ds(base, 16)]
                tbase16 = cb_t[pl.ds(base, 16)]
                gate = cb_g[pl.ds(base, 16)]
                fg16 = cb_f[pl.ds(base, 16)]
                for k in range(16):
                    yb_s = _vsplat(ybase16, k)
                    tb_s = _vsplat(tbase16, k)
                    g_s = _vsplat(gate, k)
                    fg_s = _vsplat(fg16, k)
                    gd = plsc.pack(g_s, g_s, format=plsc.PackFormat.INTERLEAVED)
                    fgd = plsc.pack(fg_s, fg_s, format=plsc.PackFormat.INTERLEAVED)
                    for w in range(NWG):
                        yw = plsc.bitcast(
                            plsc.load_gather(ytab_v, [yb_s + cols[w]]),
                            jnp.bfloat16)
                        t0 = plsc.bitcast(
                            plsc.load_gather(wtab_v, [tb_s + cols[w]]),
                            jnp.bfloat16)
                        t1 = plsc.bitcast(
                            plsc.load_gather(wtab_v, [tb_s + (HW + cols[w])]),
                            jnp.bfloat16)
                        wa = t0 * gd + (t1 - t0) * fgd
                        pr = wa * yw
                        pe, po = plsc.unpack(pr, format=plsc.PackFormat.INTERLEAVED)
                        acc[2 * w] = acc[2 * w] + pe
                        acc[2 * w + 1] = acc[2 * w + 1] + po
                return acc

            ngroups = (n_act + 15) // 16
            acc = lax.fori_loop(0, ngroups, group_body, acc)
            for w in range(2 * NWG):
                out_v[pl.ds(ai * (2 * HW) + w * 16, 16)] = acc[w]
            return carry

        lax.fori_loop(0, chunk, atom_body, 0)
        pltpu.sync_copy(
            out_v,
            out.at[pl.ds((ch * BA + crow) * (2 * HW), chunk * 2 * HW)])


def _out_body(agg_ref, x_ref, Wf2o_ref, bf2o_ref, Wd_ref, bd_ref, out_ref):
    hv = _ssp(jnp.dot(agg_ref[...].astype(jnp.bfloat16), Wf2o_ref[...],
                      preferred_element_type=jnp.float32) + bf2o_ref[...])
    v = jnp.dot(hv.astype(jnp.bfloat16), Wd_ref[...],
                preferred_element_type=jnp.float32) + bd_ref[...]
    out_ref[...] = x_ref[...] + v


def kernel(atomic_numbers, positions, neighbors, neighbor_mask, emb, W1, b1,
           W2, b2, Win2f, Wf2out, bf2out, Wdense, bdense):
    B, A, N = neighbors.shape
    d = emb.shape[1]
    ng = W1.shape[0]
    cutoff, start = 5.0, 1.2
    K = _KNOTS
    Kp = K + 8
    hu = (cutoff * cutoff) / K
    offsets_np = _np.linspace(start, cutoff, ng, dtype=_np.float32)
    width = float(offsets_np[1] - offsets_np[0])
    coeff = -0.5 / (width ** 2)
    offs_np = _np.zeros((1, 128), _np.float32)
    offs_np[0, :ng] = offsets_np
    offs_pad = jnp.asarray(offs_np)

    emb_pad = jnp.zeros((128, d), jnp.float32).at[:emb.shape[0]].set(emb)
    Ehi = emb_pad.astype(jnp.bfloat16)
    Elo = (emb_pad - Ehi.astype(jnp.float32)).astype(jnp.bfloat16)
    W1p = jnp.zeros((128, d), jnp.float32).at[:ng].set(W1)

    x, ybf = pl.pallas_call(
        _xy_body,
        grid=(B,),
        in_specs=[
            pl.BlockSpec((1, 1, A), lambda b_: (b_, 0, 0)),
            pl.BlockSpec((128, d), lambda b_: (0, 0)),
            pl.BlockSpec((128, d), lambda b_: (0, 0)),
            pl.BlockSpec((d, d), lambda b_: (0, 0)),
        ],
        out_specs=[
            pl.BlockSpec((1, A, d), lambda b_: (b_, 0, 0)),
            pl.BlockSpec((1, A, d), lambda b_: (b_, 0, 0)),
        ],
        out_shape=[
            jax.ShapeDtypeStruct((B, A, d), jnp.float32),
            jax.ShapeDtypeStruct((B, A, d), jnp.bfloat16),
        ],
    )(atomic_numbers.reshape(B, 1, A), Ehi, Elo, Win2f.astype(jnp.bfloat16))

    wtab_bf = pl.pallas_call(
        functools.partial(_tab_body, coeff=coeff, hu=hu, Kp=Kp),
        in_specs=[
            pl.BlockSpec((128, d), lambda: (0, 0)),
            pl.BlockSpec((1, d), lambda: (0, 0)),
            pl.BlockSpec((d, d), lambda: (0, 0)),
            pl.BlockSpec((1, d), lambda: (0, 0)),
            pl.BlockSpec((1, 128), lambda: (0, 0)),
        ],
        out_specs=pl.BlockSpec((Kp, d), lambda: (0, 0)),
        out_shape=jax.ShapeDtypeStruct((Kp, d), jnp.bfloat16),
    )(W1p, b1.reshape(1, d), W2, b2.reshape(1, d), offs_pad)

    ypack2 = lax.bitcast_convert_type(
        ybf.reshape(B * A, d // 2, 2), jnp.int32)
    HW = d // 4
    ypack = jnp.transpose(ypack2.reshape(B * A, 2, HW),
                          (1, 0, 2)).reshape(2 * B * A * HW)
    wtab2 = lax.bitcast_convert_type(
        wtab_bf.reshape(Kp, d // 2, 2), jnp.int32)
    wtab = jnp.transpose(wtab2.reshape(Kp, 2, HW),
                         (1, 0, 2)).reshape(2 * Kp * HW)
    posx = positions[:, :, 0].reshape(B * A)
    posy = positions[:, :, 1].reshape(B * A)
    posz = positions[:, :, 2].reshape(B * A)
    nbh_r = neighbors.reshape(B * A * N)
    mask_r = neighbor_mask.reshape(B * A * N)

    atoms_per_tile = A // 2
    chunk = 64
    NWG = 2

    sc_fn = pl.kernel(
        functools.partial(
            _sc_body, A=A, N=N, BA=B * A, K=K, inv_hu=1.0 / hu,
            atoms_per_tile=atoms_per_tile, chunk=chunk, NWG=NWG),
        out_type=jax.ShapeDtypeStruct((2 * B * A * d // 2,), jnp.float32),
        mesh=plsc.VectorSubcoreMesh(core_axis_name="c", subcore_axis_name="s",
                                    num_cores=2, num_subcores=16),
        compiler_params=pltpu.CompilerParams(needs_layout_passes=False),
        scratch_types=[
            pltpu.VMEM((A * HW,), jnp.int32),
            pltpu.VMEM((Kp * HW,), jnp.int32),
            pltpu.VMEM((A,), jnp.float32),
            pltpu.VMEM((A,), jnp.float32),
            pltpu.VMEM((A,), jnp.float32),
            pltpu.VMEM((chunk * N,), jnp.int32),
            pltpu.VMEM((chunk * N,), jnp.float32),
            pltpu.VMEM((chunk * d // 2,), jnp.float32),
            pltpu.VMEM((N + 16,), jnp.int32),
            pltpu.VMEM((N + 16,), jnp.int32),
            pltpu.VMEM((N + 16,), jnp.float32),
            pltpu.VMEM((N + 16,), jnp.float32),
        ],
    )
    y_agg_f = sc_fn(ypack, wtab, posx, posy, posz, nbh_r, mask_r)
    y_agg = jnp.concatenate(
        [y_agg_f[:B * A * d // 2].reshape(B * A, d // 2),
         y_agg_f[B * A * d // 2:].reshape(B * A, d // 2)], axis=1)

    sigma = _np.zeros(d, _np.int32)
    for w in range(d // 32):
        for j in range(16):
            sigma[32 * w + j] = 32 * w + 2 * j
            sigma[32 * w + 16 + j] = 32 * w + 2 * j + 1
    Wf2o_perm = Wf2out[jnp.asarray(sigma), :]

    RB = 1024
    out = pl.pallas_call(
        _out_body,
        grid=(B * A // RB,),
        in_specs=[
            pl.BlockSpec((RB, d), lambda i: (i, 0)),
            pl.BlockSpec((RB, d), lambda i: (i, 0)),
            pl.BlockSpec((d, d), lambda i: (0, 0)),
            pl.BlockSpec((1, d), lambda i: (0, 0)),
            pl.BlockSpec((d, d), lambda i: (0, 0)),
            pl.BlockSpec((1, d), lambda i: (0, 0)),
        ],
        out_specs=pl.BlockSpec((RB, d), lambda i: (i, 0)),
        out_shape=jax.ShapeDtypeStruct((B * A, d), jnp.float32),
    )(y_agg, x.reshape(B * A, d), Wf2o_perm.astype(jnp.bfloat16),
      bf2out.reshape(1, d), Wdense.astype(jnp.bfloat16), bdense.reshape(1, d))
    return out.reshape(B, A, d)


# speedup vs baseline: 1.0659x; 1.0659x over previous
import functools

import numpy as _np
import jax
import jax.numpy as jnp
from jax import lax
from jax.experimental import pallas as pl
from jax.experimental.pallas import tpu as pltpu
from jax.experimental.pallas import tpu_sc as plsc

_LN2 = 0.6931471805599453
_LOG2E = 1.4426950408889634
_KNOTS = 512


def _exp_poly(t):
    t = jnp.maximum(t, -87.0)
    z = t * _LOG2E
    ni = (z - 0.5).astype(jnp.int32)
    g = (z - ni.astype(jnp.float32)) * _LN2
    p = 1.0 + g * (1.0 + g * (0.5 + g * (1.0 / 6.0 + g * (
        1.0 / 24.0 + g * (1.0 / 120.0 + g * (1.0 / 720.0))))))
    scale = lax.bitcast_convert_type((ni + 127) << 23, jnp.float32)
    return p * scale


def _log1p_poly(w):
    den = 2.0 + w
    r = jnp.full_like(w, 0.4)
    for _ in range(4):
        r = r * (2.0 - den * r)
    s = w * r
    s2 = s * s
    return 2.0 * s * (1.0 + s2 * (1.0 / 3.0 + s2 * (
        1.0 / 5.0 + s2 * (1.0 / 7.0))))


def _ssp(x):
    return jnp.maximum(x, 0.0) + _log1p_poly(_exp_poly(-jnp.abs(x))) - _LN2


def _xy_body(zb_ref, Ehi_ref, Elo_ref, Wi2f_ref, W1_ref, b1_ref, W2_ref,
             b2_ref, offs_ref, x_ref, y_ref, tab_ref, *, coeff, hu, Kp):
    @pl.when(pl.program_id(0) == 0)
    def _tab():
        k = lax.broadcasted_iota(jnp.int32, (Kp, 1), 0).astype(jnp.float32)
        r = jnp.sqrt(k * hu + 1e-12)
        fg = _exp_poly(coeff * (r - offs_ref[...]) ** 2)
        h = _ssp(lax.dot_general(fg, W1_ref[...], (((1,), (0,)), ((), ())),
                                 precision=lax.Precision.HIGHEST) + b1_ref[...])
        wf = lax.dot_general(h, W2_ref[...], (((1,), (0,)), ((), ())),
                             precision=lax.Precision.HIGHEST) + b2_ref[...]
        tab_ref[...] = wf.astype(jnp.bfloat16)

    z = zb_ref[0, 0, :]
    A = z.shape[0]
    Z = (z[:, None] == lax.broadcasted_iota(jnp.int32, (A, 128), 1))
    Zb = Z.astype(jnp.bfloat16)
    x = (jnp.dot(Zb, Ehi_ref[...], preferred_element_type=jnp.float32)
         + jnp.dot(Zb, Elo_ref[...], preferred_element_type=jnp.float32))
    x_ref[0] = x
    y_ref[0] = jnp.dot(x.astype(jnp.bfloat16), Wi2f_ref[...],
                       preferred_element_type=jnp.float32).astype(jnp.bfloat16)


def _vsplat(v, k):
    idx = jnp.full((16, 1), k, dtype=jnp.int32)
    dn = lax.GatherDimensionNumbers(offset_dims=(), collapsed_slice_dims=(0,),
                                    start_index_map=(0,))
    return lax.gather(v, idx, dn, (1,),
                      mode=lax.GatherScatterMode.PROMISE_IN_BOUNDS)


def _sc_body(ypack, wtab, posx, posy, posz, nbh, mask, out, ytab_v, wtab_v,
             px_v, py_v, pz_v, nbh_v, mask_v, out_v, *, A, N, BA, K, inv_hu,
             atoms_per_tile, chunk, NWG):
    NC = 2
    cid = lax.axis_index("c")
    sid = lax.axis_index("s")
    wid = sid * NC + cid
    b = wid // 4
    sub = wid % 4
    atom0 = (sub // 2) * atoms_per_tile
    ch = sub % 2
    HW = NWG * 16

    pltpu.sync_copy(wtab.at[pl.ds(ch * ((K + 8) * HW), (K + 8) * HW)], wtab_v)
    pltpu.sync_copy(ypack.at[pl.ds((ch * BA + b * A) * HW, A * HW)], ytab_v)
    pltpu.sync_copy(posx.at[pl.ds(b * A, A)], px_v)
    pltpu.sync_copy(posy.at[pl.ds(b * A, A)], py_v)
    pltpu.sync_copy(posz.at[pl.ds(b * A, A)], pz_v)

    row0 = b * A + atom0
    iota16 = lax.broadcasted_iota(jnp.int32, (16,), 0)
    cols = [iota16 + 16 * w for w in range(NWG)]
    nq = N // 16

    for c in range(atoms_per_tile // chunk):
        crow = row0 + c * chunk
        pltpu.sync_copy(nbh.at[pl.ds(crow * N, chunk * N)], nbh_v)
        pltpu.sync_copy(mask.at[pl.ds(crow * N, chunk * N)], mask_v)

        def atom_body(ai, carry):
            a_loc = atom0 + c * chunk + ai
            af = jnp.full((16,), a_loc, dtype=jnp.int32)
            pxa = plsc.load_gather(px_v, [af])
            pya = plsc.load_gather(py_v, [af])
            pza = plsc.load_gather(pz_v, [af])
            acc = [jnp.zeros((16,), jnp.float32) for _ in range(2 * NWG)]
            for q in range(nq):
                nb16 = nbh_v[pl.ds(ai * N + q * 16, 16)]
                m16 = mask_v[pl.ds(ai * N + q * 16, 16)]
                dx = plsc.load_gather(px_v, [nb16]) - pxa
                dy = plsc.load_gather(py_v, [nb16]) - pya
                dz = plsc.load_gather(pz_v, [nb16]) - pza
                u = dx * dx + dy * dy + dz * dz
                gate = jnp.where(u + 1e-12 <= 25.0, m16, 0.0)
                t = u * inv_hu
                i016 = jnp.clip(t.astype(jnp.int32), 0, K - 1)
                fr16 = t - i016.astype(jnp.float32)
                fg16 = fr16 * gate
                ybase16 = nb16 * HW
                tbase16 = i016 * HW
                for k in range(16):
                    yb_s = _vsplat(ybase16, k)
                    tb_s = _vsplat(tbase16, k)
                    g_s = _vsplat(gate, k)
                    fg_s = _vsplat(fg16, k)
                    gd = plsc.pack(g_s, g_s, format=plsc.PackFormat.INTERLEAVED)
                    fgd = plsc.pack(fg_s, fg_s, format=plsc.PackFormat.INTERLEAVED)
                    for w in range(NWG):
                        yw = plsc.bitcast(
                            plsc.load_gather(ytab_v, [yb_s + cols[w]]),
                            jnp.bfloat16)
                        t0 = plsc.bitcast(
                            plsc.load_gather(wtab_v, [tb_s + cols[w]]),
                            jnp.bfloat16)
                        t1 = plsc.bitcast(
                            plsc.load_gather(wtab_v, [tb_s + (HW + cols[w])]),
                            jnp.bfloat16)
                        wa = t0 * gd + (t1 - t0) * fgd
                        pr = wa * yw
                        pe, po = plsc.unpack(pr, format=plsc.PackFormat.INTERLEAVED)
                        acc[2 * w] = acc[2 * w] + pe
                        acc[2 * w + 1] = acc[2 * w + 1] + po
            for w in range(2 * NWG):
                out_v[pl.ds(ai * (2 * HW) + w * 16, 16)] = acc[w]
            return carry

        lax.fori_loop(0, chunk, atom_body, 0)
        pltpu.sync_copy(
            out_v,
            out.at[pl.ds((ch * BA + crow) * (2 * HW), chunk * 2 * HW)])


def _out_body(agg_ref, x_ref, Wf2o_ref, bf2o_ref, Wd_ref, bd_ref, out_ref):
    hv = _ssp(jnp.dot(agg_ref[...].astype(jnp.bfloat16), Wf2o_ref[...],
                      preferred_element_type=jnp.float32) + bf2o_ref[...])
    v = jnp.dot(hv.astype(jnp.bfloat16), Wd_ref[...],
                preferred_element_type=jnp.float32) + bd_ref[...]
    out_ref[...] = x_ref[...] + v


def kernel(atomic_numbers, positions, neighbors, neighbor_mask, emb, W1, b1,
           W2, b2, Win2f, Wf2out, bf2out, Wdense, bdense):
    B, A, N = neighbors.shape
    d = emb.shape[1]
    ng = W1.shape[0]
    cutoff, start = 5.0, 1.2
    K = _KNOTS
    Kp = K + 8
    hu = (cutoff * cutoff) / K
    offsets_np = _np.linspace(start, cutoff, ng, dtype=_np.float32)
    width = float(offsets_np[1] - offsets_np[0])
    coeff = -0.5 / (width ** 2)
    offs_np = _np.zeros((1, 128), _np.float32)
    offs_np[0, :ng] = offsets_np
    offs_pad = jnp.asarray(offs_np)

    emb_pad = jnp.zeros((128, d), jnp.float32).at[:emb.shape[0]].set(emb)
    Ehi = emb_pad.astype(jnp.bfloat16)
    Elo = (emb_pad - Ehi.astype(jnp.float32)).astype(jnp.bfloat16)
    W1p = jnp.zeros((128, d), jnp.float32).at[:ng].set(W1)

    x, ybf, wtab_bf = pl.pallas_call(
        functools.partial(_xy_body, coeff=coeff, hu=hu, Kp=Kp),
        grid=(B,),
        in_specs=[
            pl.BlockSpec((1, 1, A), lambda b_: (b_, 0, 0)),
            pl.BlockSpec((128, d), lambda b_: (0, 0)),
            pl.BlockSpec((128, d), lambda b_: (0, 0)),
            pl.BlockSpec((d, d), lambda b_: (0, 0)),
            pl.BlockSpec((128, d), lambda b_: (0, 0)),
            pl.BlockSpec((1, d), lambda b_: (0, 0)),
            pl.BlockSpec((d, d), lambda b_: (0, 0)),
            pl.BlockSpec((1, d), lambda b_: (0, 0)),
            pl.BlockSpec((1, 128), lambda b_: (0, 0)),
        ],
        out_specs=[
            pl.BlockSpec((1, A, d), lambda b_: (b_, 0, 0)),
            pl.BlockSpec((1, A, d), lambda b_: (b_, 0, 0)),
            pl.BlockSpec((Kp, d), lambda b_: (0, 0)),
        ],
        out_shape=[
            jax.ShapeDtypeStruct((B, A, d), jnp.float32),
            jax.ShapeDtypeStruct((B, A, d), jnp.bfloat16),
            jax.ShapeDtypeStruct((Kp, d), jnp.bfloat16),
        ],
        compiler_params=pltpu.CompilerParams(
            dimension_semantics=("arbitrary",)),
    )(atomic_numbers.reshape(B, 1, A), Ehi, Elo, Win2f.astype(jnp.bfloat16),
      W1p, b1.reshape(1, d), W2, b2.reshape(1, d), offs_pad)

    ypack2 = lax.bitcast_convert_type(
        ybf.reshape(B * A, d // 2, 2), jnp.int32)
    HW = d // 4
    ypack = jnp.transpose(ypack2.reshape(B * A, 2, HW),
                          (1, 0, 2)).reshape(2 * B * A * HW)
    wtab2 = lax.bitcast_convert_type(
        wtab_bf.reshape(Kp, d // 2, 2), jnp.int32)
    wtab = jnp.transpose(wtab2.reshape(Kp, 2, HW),
                         (1, 0, 2)).reshape(2 * Kp * HW)
    posx = positions[:, :, 0].reshape(B * A)
    posy = positions[:, :, 1].reshape(B * A)
    posz = positions[:, :, 2].reshape(B * A)
    nbh_r = neighbors.reshape(B * A * N)
    mask_r = neighbor_mask.reshape(B * A * N)

    atoms_per_tile = A // 2
    chunk = 64
    NWG = 2

    sc_fn = pl.kernel(
        functools.partial(
            _sc_body, A=A, N=N, BA=B * A, K=K, inv_hu=1.0 / hu,
            atoms_per_tile=atoms_per_tile, chunk=chunk, NWG=NWG),
        out_type=jax.ShapeDtypeStruct((2 * B * A * d // 2,), jnp.float32),
        mesh=plsc.VectorSubcoreMesh(core_axis_name="c", subcore_axis_name="s",
                                    num_cores=2, num_subcores=16),
        compiler_params=pltpu.CompilerParams(needs_layout_passes=False),
        scratch_types=[
            pltpu.VMEM((A * HW,), jnp.int32),
            pltpu.VMEM((Kp * HW,), jnp.int32),
            pltpu.VMEM((A,), jnp.float32),
            pltpu.VMEM((A,), jnp.float32),
            pltpu.VMEM((A,), jnp.float32),
            pltpu.VMEM((chunk * N,), jnp.int32),
            pltpu.VMEM((chunk * N,), jnp.float32),
            pltpu.VMEM((chunk * d // 2,), jnp.float32),
        ],
    )
    y_agg_f = sc_fn(ypack, wtab, posx, posy, posz, nbh_r, mask_r)
    y_agg = jnp.concatenate(
        [y_agg_f[:B * A * d // 2].reshape(B * A, d // 2),
         y_agg_f[B * A * d // 2:].reshape(B * A, d // 2)], axis=1)

    sigma = _np.zeros(d, _np.int32)
    for w in range(d // 32):
        for j in range(16):
            sigma[32 * w + j] = 32 * w + 2 * j
            sigma[32 * w + 16 + j] = 32 * w + 2 * j + 1
    Wf2o_perm = Wf2out[jnp.asarray(sigma), :]

    RB = 1024
    out = pl.pallas_call(
        _out_body,
        grid=(B * A // RB,),
        in_specs=[
            pl.BlockSpec((RB, d), lambda i: (i, 0)),
            pl.BlockSpec((RB, d), lambda i: (i, 0)),
            pl.BlockSpec((d, d), lambda i: (0, 0)),
            pl.BlockSpec((1, d), lambda i: (0, 0)),
            pl.BlockSpec((d, d), lambda i: (0, 0)),
            pl.BlockSpec((1, d), lambda i: (0, 0)),
        ],
        out_specs=pl.BlockSpec((RB, d), lambda i: (i, 0)),
        out_shape=jax.ShapeDtypeStruct((B * A, d), jnp.float32),
    )(y_agg, x.reshape(B * A, d), Wf2o_perm.astype(jnp.bfloat16),
      bf2out.reshape(1, d), Wdense.astype(jnp.bfloat16), bdense.reshape(1, d))
    return out.reshape(B, A, d)


# speedup vs baseline: 1.2531x; 1.1756x over previous
import functools

import numpy as _np
import jax
import jax.numpy as jnp
from jax import lax
from jax.experimental import pallas as pl
from jax.experimental.pallas import tpu as pltpu
from jax.experimental.pallas import tpu_sc as plsc

_LN2 = 0.6931471805599453
_LOG2E = 1.4426950408889634
_KNOTS = 2048


def _exp_poly(t):
    t = jnp.maximum(t, -87.0)
    z = t * _LOG2E
    ni = (z - 0.5).astype(jnp.int32)
    g = (z - ni.astype(jnp.float32)) * _LN2
    p = 1.0 + g * (1.0 + g * (0.5 + g * (1.0 / 6.0 + g * (
        1.0 / 24.0 + g * (1.0 / 120.0 + g * (1.0 / 720.0))))))
    scale = lax.bitcast_convert_type((ni + 127) << 23, jnp.float32)
    return p * scale


def _log1p_poly(w):
    den = 2.0 + w
    r = jnp.full_like(w, 0.4)
    for _ in range(4):
        r = r * (2.0 - den * r)
    s = w * r
    s2 = s * s
    return 2.0 * s * (1.0 + s2 * (1.0 / 3.0 + s2 * (
        1.0 / 5.0 + s2 * (1.0 / 7.0))))


def _ssp(x):
    return jnp.maximum(x, 0.0) + _log1p_poly(_exp_poly(-jnp.abs(x))) - _LN2


def _xy_body(zb_ref, Ehi_ref, Elo_ref, Wi2f_ref, W1_ref, b1_ref, W2_ref,
             b2_ref, offs_ref, x_ref, y_ref, tab_ref, *, coeff, hu, Kp):
    @pl.when(pl.program_id(0) == 0)
    def _tab():
        k = lax.broadcasted_iota(jnp.int32, (Kp, 1), 0).astype(jnp.float32)
        r = jnp.sqrt(k * hu + 1e-12)
        fg = _exp_poly(coeff * (r - offs_ref[...]) ** 2)
        h = _ssp(lax.dot_general(fg, W1_ref[...], (((1,), (0,)), ((), ())),
                                 precision=lax.Precision.HIGHEST) + b1_ref[...])
        wf = lax.dot_general(h, W2_ref[...], (((1,), (0,)), ((), ())),
                             precision=lax.Precision.HIGHEST) + b2_ref[...]
        tab_ref[...] = wf.astype(jnp.bfloat16)

    z = zb_ref[0, 0, :]
    A = z.shape[0]
    Z = (z[:, None] == lax.broadcasted_iota(jnp.int32, (A, 128), 1))
    Zb = Z.astype(jnp.bfloat16)
    x = (jnp.dot(Zb, Ehi_ref[...], preferred_element_type=jnp.float32)
         + jnp.dot(Zb, Elo_ref[...], preferred_element_type=jnp.float32))
    x_ref[0] = x
    y_ref[0] = jnp.dot(x.astype(jnp.bfloat16), Wi2f_ref[...],
                       preferred_element_type=jnp.float32).astype(jnp.bfloat16)


def _vsplat(v, k):
    idx = jnp.full((16, 1), k, dtype=jnp.int32)
    dn = lax.GatherDimensionNumbers(offset_dims=(), collapsed_slice_dims=(0,),
                                    start_index_map=(0,))
    return lax.gather(v, idx, dn, (1,),
                      mode=lax.GatherScatterMode.PROMISE_IN_BOUNDS)


def _sc_body(ypack, wtab, posx, posy, posz, nbh, mask, out, ytab_v, wtab_v,
             px_v, py_v, pz_v, nbh_v, mask_v, out_v, *, A, N, BA, K, inv_hu,
             atoms_per_tile, chunk, NWG):
    NC = 2
    cid = lax.axis_index("c")
    sid = lax.axis_index("s")
    wid = sid * NC + cid
    b = wid // 4
    sub = wid % 4
    atom0 = (sub // 2) * atoms_per_tile
    ch = sub % 2
    HW = NWG * 16

    pltpu.sync_copy(wtab.at[pl.ds(ch * ((K + 8) * HW), (K + 8) * HW)], wtab_v)
    pltpu.sync_copy(ypack.at[pl.ds((ch * BA + b * A) * HW, A * HW)], ytab_v)
    pltpu.sync_copy(posx.at[pl.ds(b * A, A)], px_v)
    pltpu.sync_copy(posy.at[pl.ds(b * A, A)], py_v)
    pltpu.sync_copy(posz.at[pl.ds(b * A, A)], pz_v)

    row0 = b * A + atom0
    iota16 = lax.broadcasted_iota(jnp.int32, (16,), 0)
    cols = [iota16 + 16 * w for w in range(NWG)]
    nq = N // 16

    for c in range(atoms_per_tile // chunk):
        crow = row0 + c * chunk
        pltpu.sync_copy(nbh.at[pl.ds(crow * N, chunk * N)], nbh_v)
        pltpu.sync_copy(mask.at[pl.ds(crow * N, chunk * N)], mask_v)

        def atom_body(ai, carry):
            a_loc = atom0 + c * chunk + ai
            af = jnp.full((16,), a_loc, dtype=jnp.int32)
            pxa = plsc.load_gather(px_v, [af])
            pya = plsc.load_gather(py_v, [af])
            pza = plsc.load_gather(pz_v, [af])
            acc = [jnp.zeros((16,), jnp.float32) for _ in range(2 * NWG)]
            for q in range(nq):
                nb16 = nbh_v[pl.ds(ai * N + q * 16, 16)]
                m16 = mask_v[pl.ds(ai * N + q * 16, 16)]
                dx = plsc.load_gather(px_v, [nb16]) - pxa
                dy = plsc.load_gather(py_v, [nb16]) - pya
                dz = plsc.load_gather(pz_v, [nb16]) - pza
                u = dx * dx + dy * dy + dz * dz
                gate = jnp.where(u + 1e-12 <= 25.0, m16, 0.0)
                t = u * inv_hu
                i016 = jnp.clip((t + 0.5).astype(jnp.int32), 0, K)
                ybase16 = nb16 * HW
                tbase16 = i016 * HW
                for k in range(16):
                    yb_s = _vsplat(ybase16, k)
                    tb_s = _vsplat(tbase16, k)
                    g_s = _vsplat(gate, k)
                    gd = plsc.pack(g_s, g_s, format=plsc.PackFormat.INTERLEAVED)
                    for w in range(NWG):
                        yw = plsc.bitcast(
                            plsc.load_gather(ytab_v, [yb_s + cols[w]]),
                            jnp.bfloat16)
                        t0 = plsc.bitcast(
                            plsc.load_gather(wtab_v, [tb_s + cols[w]]),
                            jnp.bfloat16)
                        pr = (t0 * gd) * yw
                        pe, po = plsc.unpack(pr, format=plsc.PackFormat.INTERLEAVED)
                        acc[2 * w] = acc[2 * w] + pe
                        acc[2 * w + 1] = acc[2 * w + 1] + po
            for w in range(2 * NWG):
                out_v[pl.ds(ai * (2 * HW) + w * 16, 16)] = acc[w]
            return carry

        lax.fori_loop(0, chunk, atom_body, 0)
        pltpu.sync_copy(
            out_v,
            out.at[pl.ds((ch * BA + crow) * (2 * HW), chunk * 2 * HW)])


def _out_body(agg_ref, x_ref, Wf2o_ref, bf2o_ref, Wd_ref, bd_ref, out_ref):
    hv = _ssp(jnp.dot(agg_ref[...].astype(jnp.bfloat16), Wf2o_ref[...],
                      preferred_element_type=jnp.float32) + bf2o_ref[...])
    v = jnp.dot(hv.astype(jnp.bfloat16), Wd_ref[...],
                preferred_element_type=jnp.float32) + bd_ref[...]
    out_ref[...] = x_ref[...] + v


def kernel(atomic_numbers, positions, neighbors, neighbor_mask, emb, W1, b1,
           W2, b2, Win2f, Wf2out, bf2out, Wdense, bdense):
    B, A, N = neighbors.shape
    d = emb.shape[1]
    ng = W1.shape[0]
    cutoff, start = 5.0, 1.2
    K = _KNOTS
    Kp = K + 8
    hu = (cutoff * cutoff) / K
    offsets_np = _np.linspace(start, cutoff, ng, dtype=_np.float32)
    width = float(offsets_np[1] - offsets_np[0])
    coeff = -0.5 / (width ** 2)
    offs_np = _np.zeros((1, 128), _np.float32)
    offs_np[0, :ng] = offsets_np
    offs_pad = jnp.asarray(offs_np)

    emb_pad = jnp.zeros((128, d), jnp.float32).at[:emb.shape[0]].set(emb)
    Ehi = emb_pad.astype(jnp.bfloat16)
    Elo = (emb_pad - Ehi.astype(jnp.float32)).astype(jnp.bfloat16)
    W1p = jnp.zeros((128, d), jnp.float32).at[:ng].set(W1)

    x, ybf, wtab_bf = pl.pallas_call(
        functools.partial(_xy_body, coeff=coeff, hu=hu, Kp=Kp),
        grid=(B,),
        in_specs=[
            pl.BlockSpec((1, 1, A), lambda b_: (b_, 0, 0)),
            pl.BlockSpec((128, d), lambda b_: (0, 0)),
            pl.BlockSpec((128, d), lambda b_: (0, 0)),
            pl.BlockSpec((d, d), lambda b_: (0, 0)),
            pl.BlockSpec((128, d), lambda b_: (0, 0)),
            pl.BlockSpec((1, d), lambda b_: (0, 0)),
            pl.BlockSpec((d, d), lambda b_: (0, 0)),
            pl.BlockSpec((1, d), lambda b_: (0, 0)),
            pl.BlockSpec((1, 128), lambda b_: (0, 0)),
        ],
        out_specs=[
            pl.BlockSpec((1, A, d), lambda b_: (b_, 0, 0)),
            pl.BlockSpec((1, A, d), lambda b_: (b_, 0, 0)),
            pl.BlockSpec((Kp, d), lambda b_: (0, 0)),
        ],
        out_shape=[
            jax.ShapeDtypeStruct((B, A, d), jnp.float32),
            jax.ShapeDtypeStruct((B, A, d), jnp.bfloat16),
            jax.ShapeDtypeStruct((Kp, d), jnp.bfloat16),
        ],
        compiler_params=pltpu.CompilerParams(
            dimension_semantics=("arbitrary",)),
    )(atomic_numbers.reshape(B, 1, A), Ehi, Elo, Win2f.astype(jnp.bfloat16),
      W1p, b1.reshape(1, d), W2, b2.reshape(1, d), offs_pad)

    ypack2 = lax.bitcast_convert_type(
        ybf.reshape(B * A, d // 2, 2), jnp.int32)
    HW = d // 4
    ypack = jnp.transpose(ypack2.reshape(B * A, 2, HW),
                          (1, 0, 2)).reshape(2 * B * A * HW)
    wtab2 = lax.bitcast_convert_type(
        wtab_bf.reshape(Kp, d // 2, 2), jnp.int32)
    wtab = jnp.transpose(wtab2.reshape(Kp, 2, HW),
                         (1, 0, 2)).reshape(2 * Kp * HW)
    posx = positions[:, :, 0].reshape(B * A)
    posy = positions[:, :, 1].reshape(B * A)
    posz = positions[:, :, 2].reshape(B * A)
    nbh_r = neighbors.reshape(B * A * N)
    mask_r = neighbor_mask.reshape(B * A * N)

    atoms_per_tile = A // 2
    chunk = 64
    NWG = 2

    sc_fn = pl.kernel(
        functools.partial(
            _sc_body, A=A, N=N, BA=B * A, K=K, inv_hu=1.0 / hu,
            atoms_per_tile=atoms_per_tile, chunk=chunk, NWG=NWG),
        out_type=jax.ShapeDtypeStruct((2 * B * A * d // 2,), jnp.float32),
        mesh=plsc.VectorSubcoreMesh(core_axis_name="c", subcore_axis_name="s",
                                    num_cores=2, num_subcores=16),
        compiler_params=pltpu.CompilerParams(needs_layout_passes=False),
        scratch_types=[
            pltpu.VMEM((A * HW,), jnp.int32),
            pltpu.VMEM((Kp * HW,), jnp.int32),
            pltpu.VMEM((A,), jnp.float32),
            pltpu.VMEM((A,), jnp.float32),
            pltpu.VMEM((A,), jnp.float32),
            pltpu.VMEM((chunk * N,), jnp.int32),
            pltpu.VMEM((chunk * N,), jnp.float32),
            pltpu.VMEM((chunk * d // 2,), jnp.float32),
        ],
    )
    y_agg_f = sc_fn(ypack, wtab, posx, posy, posz, nbh_r, mask_r)
    y_agg = jnp.concatenate(
        [y_agg_f[:B * A * d // 2].reshape(B * A, d // 2),
         y_agg_f[B * A * d // 2:].reshape(B * A, d // 2)], axis=1)

    sigma = _np.zeros(d, _np.int32)
    for w in range(d // 32):
        for j in range(16):
            sigma[32 * w + j] = 32 * w + 2 * j
            sigma[32 * w + 16 + j] = 32 * w + 2 * j + 1
    Wf2o_perm = Wf2out[jnp.asarray(sigma), :]

    RB = 1024
    out = pl.pallas_call(
        _out_body,
        grid=(B * A // RB,),
        in_specs=[
            pl.BlockSpec((RB, d), lambda i: (i, 0)),
            pl.BlockSpec((RB, d), lambda i: (i, 0)),
            pl.BlockSpec((d, d), lambda i: (0, 0)),
            pl.BlockSpec((1, d), lambda i: (0, 0)),
            pl.BlockSpec((d, d), lambda i: (0, 0)),
            pl.BlockSpec((1, d), lambda i: (0, 0)),
        ],
        out_specs=pl.BlockSpec((RB, d), lambda i: (i, 0)),
        out_shape=jax.ShapeDtypeStruct((B * A, d), jnp.float32),
    )(y_agg, x.reshape(B * A, d), Wf2o_perm.astype(jnp.bfloat16),
      bf2out.reshape(1, d), Wdense.astype(jnp.bfloat16), bdense.reshape(1, d))
    return out.reshape(B, A, d)


# speedup vs baseline: 1.3593x; 1.0847x over previous
import functools

import numpy as _np
import jax
import jax.numpy as jnp
from jax import lax
from jax.experimental import pallas as pl
from jax.experimental.pallas import tpu as pltpu
from jax.experimental.pallas import tpu_sc as plsc

_LN2 = 0.6931471805599453
_LOG2E = 1.4426950408889634
_KNOTS = 2048


def _exp_poly(t):
    t = jnp.maximum(t, -87.0)
    z = t * _LOG2E
    ni = (z - 0.5).astype(jnp.int32)
    g = (z - ni.astype(jnp.float32)) * _LN2
    p = 1.0 + g * (1.0 + g * (0.5 + g * (1.0 / 6.0 + g * (
        1.0 / 24.0 + g * (1.0 / 120.0 + g * (1.0 / 720.0))))))
    scale = lax.bitcast_convert_type((ni + 127) << 23, jnp.float32)
    return p * scale


def _log1p_poly(w):
    den = 2.0 + w
    r = jnp.full_like(w, 0.4)
    for _ in range(4):
        r = r * (2.0 - den * r)
    s = w * r
    s2 = s * s
    return 2.0 * s * (1.0 + s2 * (1.0 / 3.0 + s2 * (
        1.0 / 5.0 + s2 * (1.0 / 7.0))))


def _ssp(x):
    return jnp.maximum(x, 0.0) + _log1p_poly(_exp_poly(-jnp.abs(x))) - _LN2


def _xy_body(zb_ref, Ehi_ref, Elo_ref, Wi2f_ref, W1_ref, b1_ref, W2_ref,
             b2_ref, offs_ref, x_ref, y_ref, tab_ref, *, coeff, hu, Kp):
    @pl.when(pl.program_id(0) == 0)
    def _tab():
        k = lax.broadcasted_iota(jnp.int32, (Kp, 1), 0).astype(jnp.float32)
        r = jnp.sqrt(k * hu + 1e-12)
        fg = _exp_poly(coeff * (r - offs_ref[...]) ** 2)
        h = _ssp(lax.dot_general(fg, W1_ref[...], (((1,), (0,)), ((), ())),
                                 precision=lax.Precision.HIGHEST) + b1_ref[...])
        wf = lax.dot_general(h, W2_ref[...], (((1,), (0,)), ((), ())),
                             precision=lax.Precision.HIGHEST) + b2_ref[...]
        wf = jnp.where(k <= float(Kp - 8), wf, 0.0)
        tab_ref[...] = wf.astype(jnp.bfloat16)

    z = zb_ref[0, 0, :]
    A = z.shape[0]
    Z = (z[:, None] == lax.broadcasted_iota(jnp.int32, (A, 128), 1))
    Zb = Z.astype(jnp.bfloat16)
    x = (jnp.dot(Zb, Ehi_ref[...], preferred_element_type=jnp.float32)
         + jnp.dot(Zb, Elo_ref[...], preferred_element_type=jnp.float32))
    x_ref[0] = x
    y_ref[0] = jnp.dot(x.astype(jnp.bfloat16), Wi2f_ref[...],
                       preferred_element_type=jnp.float32).astype(jnp.bfloat16)


def _vsplat(v, k):
    idx = jnp.full((16, 1), k, dtype=jnp.int32)
    dn = lax.GatherDimensionNumbers(offset_dims=(), collapsed_slice_dims=(0,),
                                    start_index_map=(0,))
    return lax.gather(v, idx, dn, (1,),
                      mode=lax.GatherScatterMode.PROMISE_IN_BOUNDS)


def _sc_body(ypack, wtab, posx, posy, posz, nbh, mask, out, ytab_v, wtab_v,
             px_v, py_v, pz_v, nbh_v, mask_v, out_v, *, A, N, BA, K, inv_hu,
             atoms_per_tile, chunk, NWG):
    NC = 2
    cid = lax.axis_index("c")
    sid = lax.axis_index("s")
    wid = sid * NC + cid
    b = wid // 4
    sub = wid % 4
    atom0 = (sub // 2) * atoms_per_tile
    ch = sub % 2
    HW = NWG * 16

    pltpu.sync_copy(wtab.at[pl.ds(ch * ((K + 8) * HW), (K + 8) * HW)], wtab_v)
    pltpu.sync_copy(ypack.at[pl.ds((ch * BA + b * A) * HW, A * HW)], ytab_v)
    pltpu.sync_copy(posx.at[pl.ds(b * A, A)], px_v)
    pltpu.sync_copy(posy.at[pl.ds(b * A, A)], py_v)
    pltpu.sync_copy(posz.at[pl.ds(b * A, A)], pz_v)

    row0 = b * A + atom0
    iota16 = lax.broadcasted_iota(jnp.int32, (16,), 0)
    cols = [iota16 + 16 * w for w in range(NWG)]
    nq = N // 16

    for c in range(atoms_per_tile // chunk):
        crow = row0 + c * chunk
        pltpu.sync_copy(nbh.at[pl.ds(crow * N, chunk * N)], nbh_v)
        pltpu.sync_copy(mask.at[pl.ds(crow * N, chunk * N)], mask_v)

        def atom_body(ai, carry):
            a_loc = atom0 + c * chunk + ai
            af = jnp.full((16,), a_loc, dtype=jnp.int32)
            pxa = plsc.load_gather(px_v, [af])
            pya = plsc.load_gather(py_v, [af])
            pza = plsc.load_gather(pz_v, [af])
            acc = [jnp.zeros((16,), jnp.float32) for _ in range(2 * NWG)]
            for q in range(nq):
                nb16 = nbh_v[pl.ds(ai * N + q * 16, 16)]
                m16 = mask_v[pl.ds(ai * N + q * 16, 16)]
                dx = plsc.load_gather(px_v, [nb16]) - pxa
                dy = plsc.load_gather(py_v, [nb16]) - pya
                dz = plsc.load_gather(pz_v, [nb16]) - pza
                u = dx * dx + dy * dy + dz * dz
                act = jnp.logical_and(u + 1e-12 <= 25.0, m16 != 0.0)
                t = u * inv_hu
                zrow = jnp.full((16,), K + 4, dtype=jnp.int32)
                i016 = jnp.where(act, jnp.clip((t + 0.5).astype(jnp.int32), 0, K),
                                 zrow)
                ybase16 = nb16 * HW
                tbase16 = i016 * HW
                for k in range(16):
                    yb_s = _vsplat(ybase16, k)
                    tb_s = _vsplat(tbase16, k)
                    for w in range(NWG):
                        yw = plsc.bitcast(
                            plsc.load_gather(ytab_v, [yb_s + cols[w]]),
                            jnp.bfloat16)
                        t0 = plsc.bitcast(
                            plsc.load_gather(wtab_v, [tb_s + cols[w]]),
                            jnp.bfloat16)
                        pr = t0 * yw
                        pe, po = plsc.unpack(pr, format=plsc.PackFormat.INTERLEAVED)
                        acc[2 * w] = acc[2 * w] + pe
                        acc[2 * w + 1] = acc[2 * w + 1] + po
            for w in range(2 * NWG):
                out_v[pl.ds(ai * (2 * HW) + w * 16, 16)] = acc[w]
            return carry

        lax.fori_loop(0, chunk, atom_body, 0)
        pltpu.sync_copy(
            out_v,
            out.at[pl.ds((ch * BA + crow) * (2 * HW), chunk * 2 * HW)])


def _out_body(agg_ref, x_ref, Wf2o_ref, bf2o_ref, Wd_ref, bd_ref, out_ref):
    hv = _ssp(jnp.dot(agg_ref[...].astype(jnp.bfloat16), Wf2o_ref[...],
                      preferred_element_type=jnp.float32) + bf2o_ref[...])
    v = jnp.dot(hv.astype(jnp.bfloat16), Wd_ref[...],
                preferred_element_type=jnp.float32) + bd_ref[...]
    out_ref[...] = x_ref[...] + v


def kernel(atomic_numbers, positions, neighbors, neighbor_mask, emb, W1, b1,
           W2, b2, Win2f, Wf2out, bf2out, Wdense, bdense):
    B, A, N = neighbors.shape
    d = emb.shape[1]
    ng = W1.shape[0]
    cutoff, start = 5.0, 1.2
    K = _KNOTS
    Kp = K + 8
    hu = (cutoff * cutoff) / K
    offsets_np = _np.linspace(start, cutoff, ng, dtype=_np.float32)
    width = float(offsets_np[1] - offsets_np[0])
    coeff = -0.5 / (width ** 2)
    offs_np = _np.zeros((1, 128), _np.float32)
    offs_np[0, :ng] = offsets_np
    offs_pad = jnp.asarray(offs_np)

    emb_pad = jnp.zeros((128, d), jnp.float32).at[:emb.shape[0]].set(emb)
    Ehi = emb_pad.astype(jnp.bfloat16)
    Elo = (emb_pad - Ehi.astype(jnp.float32)).astype(jnp.bfloat16)
    W1p = jnp.zeros((128, d), jnp.float32).at[:ng].set(W1)

    x, ybf, wtab_bf = pl.pallas_call(
        functools.partial(_xy_body, coeff=coeff, hu=hu, Kp=Kp),
        grid=(B,),
        in_specs=[
            pl.BlockSpec((1, 1, A), lambda b_: (b_, 0, 0)),
            pl.BlockSpec((128, d), lambda b_: (0, 0)),
            pl.BlockSpec((128, d), lambda b_: (0, 0)),
            pl.BlockSpec((d, d), lambda b_: (0, 0)),
            pl.BlockSpec((128, d), lambda b_: (0, 0)),
            pl.BlockSpec((1, d), lambda b_: (0, 0)),
            pl.BlockSpec((d, d), lambda b_: (0, 0)),
            pl.BlockSpec((1, d), lambda b_: (0, 0)),
            pl.BlockSpec((1, 128), lambda b_: (0, 0)),
        ],
        out_specs=[
            pl.BlockSpec((1, A, d), lambda b_: (b_, 0, 0)),
            pl.BlockSpec((1, A, d), lambda b_: (b_, 0, 0)),
            pl.BlockSpec((Kp, d), lambda b_: (0, 0)),
        ],
        out_shape=[
            jax.ShapeDtypeStruct((B, A, d), jnp.float32),
            jax.ShapeDtypeStruct((B, A, d), jnp.bfloat16),
            jax.ShapeDtypeStruct((Kp, d), jnp.bfloat16),
        ],
        compiler_params=pltpu.CompilerParams(
            dimension_semantics=("arbitrary",)),
    )(atomic_numbers.reshape(B, 1, A), Ehi, Elo, Win2f.astype(jnp.bfloat16),
      W1p, b1.reshape(1, d), W2, b2.reshape(1, d), offs_pad)

    ypack2 = lax.bitcast_convert_type(
        ybf.reshape(B * A, d // 2, 2), jnp.int32)
    HW = d // 4
    ypack = jnp.transpose(ypack2.reshape(B * A, 2, HW),
                          (1, 0, 2)).reshape(2 * B * A * HW)
    wtab2 = lax.bitcast_convert_type(
        wtab_bf.reshape(Kp, d // 2, 2), jnp.int32)
    wtab = jnp.transpose(wtab2.reshape(Kp, 2, HW),
                         (1, 0, 2)).reshape(2 * Kp * HW)
    posx = positions[:, :, 0].reshape(B * A)
    posy = positions[:, :, 1].reshape(B * A)
    posz = positions[:, :, 2].reshape(B * A)
    nbh_r = neighbors.reshape(B * A * N)
    mask_r = neighbor_mask.reshape(B * A * N)

    atoms_per_tile = A // 2
    chunk = 64
    NWG = 2

    sc_fn = pl.kernel(
        functools.partial(
            _sc_body, A=A, N=N, BA=B * A, K=K, inv_hu=1.0 / hu,
            atoms_per_tile=atoms_per_tile, chunk=chunk, NWG=NWG),
        out_type=jax.ShapeDtypeStruct((2 * B * A * d // 2,), jnp.float32),
        mesh=plsc.VectorSubcoreMesh(core_axis_name="c", subcore_axis_name="s",
                                    num_cores=2, num_subcores=16),
        compiler_params=pltpu.CompilerParams(needs_layout_passes=False),
        scratch_types=[
            pltpu.VMEM((A * HW,), jnp.int32),
            pltpu.VMEM((Kp * HW,), jnp.int32),
            pltpu.VMEM((A,), jnp.float32),
            pltpu.VMEM((A,), jnp.float32),
            pltpu.VMEM((A,), jnp.float32),
            pltpu.VMEM((chunk * N,), jnp.int32),
            pltpu.VMEM((chunk * N,), jnp.float32),
            pltpu.VMEM((chunk * d // 2,), jnp.float32),
        ],
    )
    y_agg_f = sc_fn(ypack, wtab, posx, posy, posz, nbh_r, mask_r)
    y_agg = jnp.concatenate(
        [y_agg_f[:B * A * d // 2].reshape(B * A, d // 2),
         y_agg_f[B * A * d // 2:].reshape(B * A, d // 2)], axis=1)

    sigma = _np.zeros(d, _np.int32)
    for w in range(d // 32):
        for j in range(16):
            sigma[32 * w + j] = 32 * w + 2 * j
            sigma[32 * w + 16 + j] = 32 * w + 2 * j + 1
    Wf2o_perm = Wf2out[jnp.asarray(sigma), :]

    RB = 1024
    out = pl.pallas_call(
        _out_body,
        grid=(B * A // RB,),
        in_specs=[
            pl.BlockSpec((RB, d), lambda i: (i, 0)),
            pl.BlockSpec((RB, d), lambda i: (i, 0)),
            pl.BlockSpec((d, d), lambda i: (0, 0)),
            pl.BlockSpec((1, d), lambda i: (0, 0)),
            pl.BlockSpec((d, d), lambda i: (0, 0)),
            pl.BlockSpec((1, d), lambda i: (0, 0)),
        ],
        out_specs=pl.BlockSpec((RB, d), lambda i: (i, 0)),
        out_shape=jax.ShapeDtypeStruct((B * A, d), jnp.float32),
    )(y_agg, x.reshape(B * A, d), Wf2o_perm.astype(jnp.bfloat16),
      bf2out.reshape(1, d), Wdense.astype(jnp.bfloat16), bdense.reshape(1, d))
    return out.reshape(B, A, d)


# speedup vs baseline: 1.3968x; 1.0276x over previous
import functools

import numpy as _np
import jax
import jax.numpy as jnp
from jax import lax
from jax.experimental import pallas as pl
from jax.experimental.pallas import tpu as pltpu
from jax.experimental.pallas import tpu_sc as plsc

_LN2 = 0.6931471805599453
_LOG2E = 1.4426950408889634
_KNOTS = 2048


def _exp_poly(t):
    t = jnp.maximum(t, -87.0)
    z = t * _LOG2E
    ni = (z - 0.5).astype(jnp.int32)
    g = (z - ni.astype(jnp.float32)) * _LN2
    p = 1.0 + g * (1.0 + g * (0.5 + g * (1.0 / 6.0 + g * (
        1.0 / 24.0 + g * (1.0 / 120.0 + g * (1.0 / 720.0))))))
    scale = lax.bitcast_convert_type((ni + 127) << 23, jnp.float32)
    return p * scale


def _log1p_poly(w):
    den = 2.0 + w
    r = jnp.full_like(w, 0.4)
    for _ in range(4):
        r = r * (2.0 - den * r)
    s = w * r
    s2 = s * s
    return 2.0 * s * (1.0 + s2 * (1.0 / 3.0 + s2 * (
        1.0 / 5.0 + s2 * (1.0 / 7.0))))


def _ssp(x):
    return jnp.maximum(x, 0.0) + _log1p_poly(_exp_poly(-jnp.abs(x))) - _LN2


def _xy_body(zb_ref, Ehi_ref, Elo_ref, Wi2f_ref, W1_ref, b1_ref, W2_ref,
             b2_ref, offs_ref, x_ref, y_ref, tab_ref, *, coeff, hu, Kp):
    @pl.when(pl.program_id(0) == 0)
    def _tab():
        k = lax.broadcasted_iota(jnp.int32, (Kp, 1), 0).astype(jnp.float32)
        r = jnp.sqrt(k * hu + 1e-12)
        fg = _exp_poly(coeff * (r - offs_ref[...]) ** 2)
        h = _ssp(lax.dot_general(fg, W1_ref[...], (((1,), (0,)), ((), ())),
                                 precision=lax.Precision.HIGHEST) + b1_ref[...])
        wf = lax.dot_general(h, W2_ref[...], (((1,), (0,)), ((), ())),
                             precision=lax.Precision.HIGHEST) + b2_ref[...]
        wf = jnp.where(k <= float(Kp - 8), wf, 0.0)
        tab_ref[...] = wf.astype(jnp.bfloat16)

    z = zb_ref[0, 0, :]
    A = z.shape[0]
    Z = (z[:, None] == lax.broadcasted_iota(jnp.int32, (A, 128), 1))
    Zb = Z.astype(jnp.bfloat16)
    x = (jnp.dot(Zb, Ehi_ref[...], preferred_element_type=jnp.float32)
         + jnp.dot(Zb, Elo_ref[...], preferred_element_type=jnp.float32))
    x_ref[0] = x
    y_ref[0] = jnp.dot(x.astype(jnp.bfloat16), Wi2f_ref[...],
                       preferred_element_type=jnp.float32).astype(jnp.bfloat16)


def _vsplat(v, k):
    idx = jnp.full((16, 1), k, dtype=jnp.int32)
    dn = lax.GatherDimensionNumbers(offset_dims=(), collapsed_slice_dims=(0,),
                                    start_index_map=(0,))
    return lax.gather(v, idx, dn, (1,),
                      mode=lax.GatherScatterMode.PROMISE_IN_BOUNDS)


def _sc_body(ypack, wtab, posx, posy, posz, nbh, mask, out, ytab_v, wtab_v,
             px_v, py_v, pz_v, nbh_v, mask_v, out_v, *, A, N, BA, K, inv_hu,
             atoms_per_tile, chunk, NWG):
    NC = 2
    cid = lax.axis_index("c")
    sid = lax.axis_index("s")
    wid = sid * NC + cid
    b = wid // 4
    sub = wid % 4
    atom0 = (sub // 2) * atoms_per_tile
    ch = sub % 2
    HW = NWG * 16

    pltpu.sync_copy(wtab.at[pl.ds(ch * ((K + 8) * HW), (K + 8) * HW)], wtab_v)
    pltpu.sync_copy(ypack.at[pl.ds((ch * BA + b * A) * HW, A * HW)], ytab_v)
    pltpu.sync_copy(posx.at[pl.ds(b * A, A)], px_v)
    pltpu.sync_copy(posy.at[pl.ds(b * A, A)], py_v)
    pltpu.sync_copy(posz.at[pl.ds(b * A, A)], pz_v)

    row0 = b * A + atom0
    iota16 = lax.broadcasted_iota(jnp.int32, (16,), 0)
    cols = [iota16 + 16 * w for w in range(NWG)]
    nq = N // 16

    for c in range(atoms_per_tile // chunk):
        crow = row0 + c * chunk
        pltpu.sync_copy(nbh.at[pl.ds(crow * N, chunk * N)], nbh_v)
        pltpu.sync_copy(mask.at[pl.ds(crow * N, chunk * N)], mask_v)

        def atom_body(ai, carry):
            a_loc = atom0 + c * chunk + ai
            af = jnp.full((16,), a_loc, dtype=jnp.int32)
            pxa = plsc.load_gather(px_v, [af])
            pya = plsc.load_gather(py_v, [af])
            pza = plsc.load_gather(pz_v, [af])
            acc = [jnp.zeros((16,), jnp.float32) for _ in range(2 * NWG)]
            for q in range(nq):
                nb16 = nbh_v[pl.ds(ai * N + q * 16, 16)]
                m16 = mask_v[pl.ds(ai * N + q * 16, 16)]
                dx = plsc.load_gather(px_v, [nb16]) - pxa
                dy = plsc.load_gather(py_v, [nb16]) - pya
                dz = plsc.load_gather(pz_v, [nb16]) - pza
                u = dx * dx + dy * dy + dz * dz
                act = jnp.logical_and(u + 1e-12 <= 25.0, m16 != 0.0)
                t = u * inv_hu
                zrow = jnp.full((16,), K + 4, dtype=jnp.int32)
                i016 = jnp.where(act, jnp.clip((t + 0.5).astype(jnp.int32), 0, K),
                                 zrow)
                ybase16 = nb16 * HW
                tbase16 = i016 * HW
                for k in range(16):
                    yb_s = _vsplat(ybase16, k)
                    tb_s = _vsplat(tbase16, k)
                    for w in range(NWG):
                        yw = plsc.bitcast(
                            plsc.load_gather(ytab_v, [yb_s + cols[w]]),
                            jnp.bfloat16)
                        t0 = plsc.bitcast(
                            plsc.load_gather(wtab_v, [tb_s + cols[w]]),
                            jnp.bfloat16)
                        pr = t0 * yw
                        pe, po = plsc.unpack(pr, format=plsc.PackFormat.INTERLEAVED)
                        acc[2 * w] = acc[2 * w] + pe
                        acc[2 * w + 1] = acc[2 * w + 1] + po
            for w in range(2 * NWG):
                out_v[pl.ds(ai * (2 * HW) + w * 16, 16)] = acc[w]
            return carry

        lax.fori_loop(0, chunk, atom_body, 0)
        pltpu.sync_copy(
            out_v,
            out.at[pl.ds((ch * BA + crow) * (2 * HW), chunk * 2 * HW)])


def _out_body(agg_ref, x_ref, Wf2o_ref, bf2o_ref, Wd_ref, bd_ref, out_ref):
    hv = _ssp(jnp.dot(agg_ref[...].astype(jnp.bfloat16), Wf2o_ref[...],
                      preferred_element_type=jnp.float32) + bf2o_ref[...])
    v = jnp.dot(hv.astype(jnp.bfloat16), Wd_ref[...],
                preferred_element_type=jnp.float32) + bd_ref[...]
    out_ref[...] = x_ref[...] + v


def kernel(atomic_numbers, positions, neighbors, neighbor_mask, emb, W1, b1,
           W2, b2, Win2f, Wf2out, bf2out, Wdense, bdense):
    B, A, N = neighbors.shape
    d = emb.shape[1]
    ng = W1.shape[0]
    cutoff, start = 5.0, 1.2
    K = _KNOTS
    Kp = K + 8
    hu = (cutoff * cutoff) / K
    offsets_np = _np.linspace(start, cutoff, ng, dtype=_np.float32)
    width = float(offsets_np[1] - offsets_np[0])
    coeff = -0.5 / (width ** 2)
    offs_np = _np.zeros((1, 128), _np.float32)
    offs_np[0, :ng] = offsets_np
    offs_pad = jnp.asarray(offs_np)

    emb_pad = jnp.zeros((128, d), jnp.float32).at[:emb.shape[0]].set(emb)
    Ehi = emb_pad.astype(jnp.bfloat16)
    Elo = (emb_pad - Ehi.astype(jnp.float32)).astype(jnp.bfloat16)
    W1p = jnp.zeros((128, d), jnp.float32).at[:ng].set(W1)

    x, ybf, wtab_bf = pl.pallas_call(
        functools.partial(_xy_body, coeff=coeff, hu=hu, Kp=Kp),
        grid=(B,),
        in_specs=[
            pl.BlockSpec((1, 1, A), lambda b_: (b_, 0, 0)),
            pl.BlockSpec((128, d), lambda b_: (0, 0)),
            pl.BlockSpec((128, d), lambda b_: (0, 0)),
            pl.BlockSpec((d, d), lambda b_: (0, 0)),
            pl.BlockSpec((128, d), lambda b_: (0, 0)),
            pl.BlockSpec((1, d), lambda b_: (0, 0)),
            pl.BlockSpec((d, d), lambda b_: (0, 0)),
            pl.BlockSpec((1, d), lambda b_: (0, 0)),
            pl.BlockSpec((1, 128), lambda b_: (0, 0)),
        ],
        out_specs=[
            pl.BlockSpec((1, A, d), lambda b_: (b_, 0, 0)),
            pl.BlockSpec((1, A, d), lambda b_: (b_, 0, 0)),
            pl.BlockSpec((Kp, d), lambda b_: (0, 0)),
        ],
        out_shape=[
            jax.ShapeDtypeStruct((B, A, d), jnp.float32),
            jax.ShapeDtypeStruct((B, A, d), jnp.bfloat16),
            jax.ShapeDtypeStruct((Kp, d), jnp.bfloat16),
        ],
        compiler_params=pltpu.CompilerParams(
            dimension_semantics=("arbitrary",)),
    )(atomic_numbers.reshape(B, 1, A), Ehi, Elo, Win2f.astype(jnp.bfloat16),
      W1p, b1.reshape(1, d), W2, b2.reshape(1, d), offs_pad)

    ypack2 = lax.bitcast_convert_type(
        ybf.reshape(B * A, d // 2, 2), jnp.int32)
    HW = d // 4
    ypack = jnp.transpose(ypack2.reshape(B * A, 2, HW),
                          (1, 0, 2)).reshape(2 * B * A * HW)
    wtab2 = lax.bitcast_convert_type(
        wtab_bf.reshape(Kp, d // 2, 2), jnp.int32)
    wtab = jnp.transpose(wtab2.reshape(Kp, 2, HW),
                         (1, 0, 2)).reshape(2 * Kp * HW)
    posx = positions[:, :, 0].reshape(B * A)
    posy = positions[:, :, 1].reshape(B * A)
    posz = positions[:, :, 2].reshape(B * A)
    nbh_r = neighbors.reshape(B * A * N)
    mask_r = neighbor_mask.reshape(B * A * N)

    atoms_per_tile = A // 2
    chunk = 128
    NWG = 2

    sc_fn = pl.kernel(
        functools.partial(
            _sc_body, A=A, N=N, BA=B * A, K=K, inv_hu=1.0 / hu,
            atoms_per_tile=atoms_per_tile, chunk=chunk, NWG=NWG),
        out_type=jax.ShapeDtypeStruct((2 * B * A * d // 2,), jnp.float32),
        mesh=plsc.VectorSubcoreMesh(core_axis_name="c", subcore_axis_name="s",
                                    num_cores=2, num_subcores=16),
        compiler_params=pltpu.CompilerParams(needs_layout_passes=False),
        scratch_types=[
            pltpu.VMEM((A * HW,), jnp.int32),
            pltpu.VMEM((Kp * HW,), jnp.int32),
            pltpu.VMEM((A,), jnp.float32),
            pltpu.VMEM((A,), jnp.float32),
            pltpu.VMEM((A,), jnp.float32),
            pltpu.VMEM((chunk * N,), jnp.int32),
            pltpu.VMEM((chunk * N,), jnp.float32),
            pltpu.VMEM((chunk * d // 2,), jnp.float32),
        ],
    )
    y_agg_f = sc_fn(ypack, wtab, posx, posy, posz, nbh_r, mask_r)
    y_agg = jnp.concatenate(
        [y_agg_f[:B * A * d // 2].reshape(B * A, d // 2),
         y_agg_f[B * A * d // 2:].reshape(B * A, d // 2)], axis=1)

    sigma = _np.zeros(d, _np.int32)
    for w in range(d // 32):
        for j in range(16):
            sigma[32 * w + j] = 32 * w + 2 * j
            sigma[32 * w + 16 + j] = 32 * w + 2 * j + 1
    Wf2o_perm = Wf2out[jnp.asarray(sigma), :]

    RB = 1024
    out = pl.pallas_call(
        _out_body,
        grid=(B * A // RB,),
        in_specs=[
            pl.BlockSpec((RB, d), lambda i: (i, 0)),
            pl.BlockSpec((RB, d), lambda i: (i, 0)),
            pl.BlockSpec((d, d), lambda i: (0, 0)),
            pl.BlockSpec((1, d), lambda i: (0, 0)),
            pl.BlockSpec((d, d), lambda i: (0, 0)),
            pl.BlockSpec((1, d), lambda i: (0, 0)),
        ],
        out_specs=pl.BlockSpec((RB, d), lambda i: (i, 0)),
        out_shape=jax.ShapeDtypeStruct((B * A, d), jnp.float32),
    )(y_agg, x.reshape(B * A, d), Wf2o_perm.astype(jnp.bfloat16),
      bf2out.reshape(1, d), Wdense.astype(jnp.bfloat16), bdense.reshape(1, d))
    return out.reshape(B, A, d)
